# baseline (speedup 1.0000x reference)
import functools

import jax
import jax.numpy as jnp
from jax import lax
from jax.experimental import pallas as pl
from jax.experimental.pallas import tpu as pltpu
from jax.experimental.pallas import tpu_sc as plsc

_VOCAB = 1000
_DEC = 512
_NFEAT_DEC = 448
_B = 4096
_U = 50
_NCHUNK = 8
_CW = 64
_NGROUP = 4
_GB = _B // _NGROUP
_BBLK = 8
_NROW = _BBLK * _U
_NB = _GB // _BBLK
_NV = _NROW // 16


def _sc_decoder(yflat, wdec_r, wswit, ab8, out, ab_v, yraw_v, idx_v,
                rows_v, out_v, sem_ab, sem_y, sem_g, sem_o):
    cid = lax.axis_index("c")
    sid = lax.axis_index("s")
    wid = sid * 2 + cid
    chunk = wid % _NCHUNK
    bg = wid // _NCHUNK
    b_base = bg * _GB
    is_dec = chunk < _NCHUNK - 1

    acp = pltpu.async_copy(ab8.at[chunk], ab_v, sem_ab)

    def start_yraw(nb, slot):
        e0 = (b_base + nb * _BBLK) * _U * 2
        pltpu.async_copy(yflat.at[pl.ds(e0, 2 * _NROW)], yraw_v.at[slot],
                         sem_y.at[slot])

    def wait_yraw(slot):
        pltpu.make_async_copy(yflat.at[pl.ds(0, 2 * _NROW)],
                              yraw_v.at[slot], sem_y.at[slot]).wait()

    def start_gather(slot):
        @pl.when(is_dec)
        def _():
            pltpu.async_copy(wdec_r.at[idx_v.at[slot]], rows_v.at[slot],
                             sem_g.at[slot])

        @pl.when(jnp.logical_not(is_dec))
        def _():
            pltpu.async_copy(wswit.at[idx_v.at[slot]], rows_v.at[slot],
                             sem_g.at[slot])

    def wait_gather(slot):
        pltpu.make_async_copy(wswit.at[idx_v.at[slot]], rows_v.at[slot],
                              sem_g.at[slot]).wait()

    def wait_out(slot):
        pltpu.make_async_copy(
            out_v.at[slot],
            out.at[pl.ds(b_base, _BBLK), :, pl.ds(chunk * _CW, _CW)],
            sem_o.at[slot]).wait()

    iota = lax.iota(jnp.int32, 16)
    col = jnp.where(is_dec, 0, 1)
    mult = jnp.where(is_dec, 7, 1)
    addc = jnp.where(is_dec, chunk, 0)
    base_v = 2 * iota + jnp.full((16,), col, jnp.int32)
    mult_v = jnp.full((16,), mult, jnp.int32)
    addc_v = jnp.full((16,), addc, jnp.int32)

    def compact(slot):
        slot_v = jnp.full((16,), slot, jnp.int32)
        for k in range(_NV):
            g = plsc.load_gather(yraw_v, [slot_v, base_v + 32 * k])
            idx_v[slot, pl.ds(16 * k, 16)] = g * mult_v + addc_v

    start_yraw(0, 0)
    start_yraw(1, 1)
    wait_yraw(0)
    compact(0)
    start_gather(0)
    acp.wait()

    a_regs = [ab_v[0, pl.ds(16 * s, 16)] for s in range(4)]
    b_regs = [ab_v[1, pl.ds(16 * s, 16)] for s in range(4)]
    zeros = jnp.zeros((16,), jnp.float32)
    half = _U // 2

    def nb_body(nb, carry):
        slot = nb % 2
        nslot = (nb + 1) % 2

        wait_gather(slot)

        @pl.when(nb + 2 < _NB)
        def _():
            start_yraw(nb + 2, slot)

        @pl.when(nb + 1 < _NB)
        def _():
            wait_yraw(nslot)
            compact(nslot)
            start_gather(nslot)

        @pl.when(nb >= 2)
        def _():
            wait_out(slot)

        @plsc.parallel_loop(0, _BBLK)
        def bi_body(bi):
            r = bi * _U
            prev_a = [zeros, zeros, zeros, zeros]
            prev_b = [rows_v[slot, r + half - 1, pl.ds(16 * s, 16)]
                      for s in range(4)]
            for u in range(half):
                cur_a = [rows_v[slot, r + u, pl.ds(16 * s, 16)]
                         for s in range(4)]
                cur_b = [rows_v[slot, r + half + u, pl.ds(16 * s, 16)]
                         for s in range(4)]
                oa = [jnp.maximum(a_regs[s] * prev_a[s]
                                  + b_regs[s] * cur_a[s], 0.0)
                      for s in range(4)]
                ob = [jnp.maximum(a_regs[s] * prev_b[s]
                                  + b_regs[s] * cur_b[s], 0.0)
                      for s in range(4)]
                for s in range(4):
                    out_v[slot, bi, u, pl.ds(16 * s, 16)] = oa[s]
                for s in range(4):
                    out_v[slot, bi, half + u, pl.ds(16 * s, 16)] = ob[s]
                prev_a = cur_a
                prev_b = cur_b

        b0 = b_base + nb * _BBLK
        pltpu.async_copy(
            out_v.at[slot],
            out.at[pl.ds(b0, _BBLK), :, pl.ds(chunk * _CW, _CW)],
            sem_o.at[slot])
        return carry

    lax.fori_loop(0, _NB, nb_body, 0)
    wait_out(0)
    wait_out(1)


_sc_call = functools.partial(
    pl.kernel,
    mesh=plsc.VectorSubcoreMesh(core_axis_name="c", subcore_axis_name="s"),
    out_type=jax.ShapeDtypeStruct((_B, _U, _DEC), jnp.float32),
    scratch_types=[
        pltpu.VMEM((2, _CW), jnp.float32),
        pltpu.VMEM((2, 2 * _NROW), jnp.int32),
        pltpu.VMEM((2, _NROW), jnp.int32),
        pltpu.VMEM((2, _NROW, _CW), jnp.float32),
        pltpu.VMEM((2, _BBLK, _U, _CW), jnp.float32),
        pltpu.SemaphoreType.DMA,
        pltpu.SemaphoreType.DMA((2,)),
        pltpu.SemaphoreType.DMA((2,)),
        pltpu.SemaphoreType.DMA((2,)),
    ],
    compiler_params=pltpu.CompilerParams(use_tc_tiling_on_sc=False,
                                         needs_layout_passes=False),
)(_sc_decoder)


def kernel(y, W_dec, W_swit, s_dec, s_swit, conv_w, conv_s):
    y = y.astype(jnp.int32)
    yflat = y.reshape(-1)
    wdec_r = W_dec.reshape(_VOCAB * (_NCHUNK - 1), _CW)
    escale = jnp.concatenate([
        jnp.full((_NFEAT_DEC,), jnp.exp(s_dec), jnp.float32),
        jnp.full((_DEC - _NFEAT_DEC,), jnp.exp(s_swit), jnp.float32),
    ])
    wscale = jnp.exp(conv_s) * escale
    a_tap = conv_w[:, 0, 0] * wscale
    b_tap = conv_w[:, 0, 1] * wscale
    ab8 = jnp.stack([a_tap, b_tap], 0).reshape(2, _NCHUNK, _CW)
    ab8 = ab8.transpose(1, 0, 2)
    return _sc_call(yflat, wdec_r, W_swit, ab8)
